# baseline (speedup 1.0000x reference)
import jax
import jax.numpy as jnp
from jax import lax
from jax.experimental import pallas as pl
from jax.experimental.pallas import tpu as pltpu
from jax.experimental.pallas import tpu_sc as plsc

N = 10000
D = 256
H = 128
NCORE = 2
NSUB = 16
CH = 112
NCH = 90
U = 6
NB_ROWS = 3
E_PAD = NSUB * NCH * CH
N_PAD = 10112
ROWS_PT = N_PAD // NSUB
OUT_PT = 632


def _mm1_body(x_ref, w_ref, o_ref):
    x = x_ref[...]
    o_ref[:, 0, :] = jnp.dot(x, w_ref[:, :H], preferred_element_type=jnp.float32)
    o_ref[:, 1, :] = jnp.dot(x, w_ref[:, H:], preferred_element_type=jnp.float32)


def _mm2_body(h_ref, w_ref, o_ref):
    a0 = jnp.maximum(h_ref[:, 0, :], 0.0)
    a1 = jnp.maximum(h_ref[:, 1, :], 0.0)
    o_ref[:, 0, :] = (
        jnp.dot(a0, w_ref[:H, :H], preferred_element_type=jnp.float32)
        + jnp.dot(a1, w_ref[H:, :H], preferred_element_type=jnp.float32))
    o_ref[:, 1, :] = (
        jnp.dot(a0, w_ref[:H, H:], preferred_element_type=jnp.float32)
        + jnp.dot(a1, w_ref[H:, H:], preferred_element_type=jnp.float32))


_RB = 1000
_NB = N // _RB

_mm1 = pl.pallas_call(
    _mm1_body,
    grid=(_NB,),
    in_specs=[
        pl.BlockSpec((_RB, D), lambda i: (i, 0)),
        pl.BlockSpec((D, D), lambda i: (0, 0)),
    ],
    out_specs=pl.BlockSpec((_RB, NCORE, H), lambda i: (i, 0, 0)),
    out_shape=jax.ShapeDtypeStruct((N, NCORE, H), jnp.float32),
)

_mm2 = pl.pallas_call(
    _mm2_body,
    grid=(_NB,),
    in_specs=[
        pl.BlockSpec((_RB, NCORE, H), lambda i: (i, 0, 0)),
        pl.BlockSpec((D, D), lambda i: (0, 0)),
    ],
    out_specs=pl.BlockSpec((_RB, NCORE, H), lambda i: (i, 0, 0)),
    out_shape=jax.ShapeDtypeStruct((N, NCORE, H), jnp.float32),
)


def _agg_body(t_hbm, src_hbm, dst_hbm, b_hbm, out_hbm,
              idx_v, rows_v, acc_sh, gsem0, gsem1, gsem2, ssem0, ssem1, ssem2):
    gsem = (gsem0, gsem1, gsem2)
    ssem = (ssem0, ssem1, ssem2)
    c = lax.axis_index("c")
    s = lax.axis_index("s")

    pltpu.sync_copy(b_hbm.at[c], rows_v.at[0])
    for k in range(ROWS_PT // CH):
        pltpu.sync_copy(rows_v.at[0],
                        acc_sh.at[pl.ds(s * ROWS_PT + k * CH, CH)])
    _rem = ROWS_PT % CH
    if _rem:
        pltpu.sync_copy(
            rows_v.at[0, pl.ds(0, _rem)],
            acc_sh.at[pl.ds(s * ROWS_PT + (ROWS_PT // CH) * CH, _rem)])
    plsc.subcore_barrier()

    def step(jj, carry):
        pltpu.sync_copy(src_hbm.at[s, jj], idx_v.at[0])
        pltpu.sync_copy(dst_hbm.at[s, jj], idx_v.at[1])

        @pl.when(c == 1)
        def _bump():
            for u in range(U):
                for k in range(CH // 16):
                    idx_v[0, u, pl.ds(k * 16, 16)] = (
                        idx_v[0, u, pl.ds(k * 16, 16)] + 1)

        gd = [None] * U
        sd = [None] * U
        gd[0] = pltpu.async_copy(t_hbm.at[idx_v.at[0, 0]], rows_v.at[0],
                                 gsem[0])
        gd[1] = pltpu.async_copy(t_hbm.at[idx_v.at[0, 1]], rows_v.at[1],
                                 gsem[1])
        for u in range(U):
            b = u % NB_ROWS
            gd[u].wait()
            sd[u] = pltpu.async_copy(rows_v.at[b], acc_sh.at[idx_v.at[1, u]],
                                     ssem[b], add=True)
            if u + 2 < U:
                if u >= 1:
                    sd[u - 1].wait()
                b2 = (u + 2) % NB_ROWS
                gd[u + 2] = pltpu.async_copy(
                    t_hbm.at[idx_v.at[0, u + 2]], rows_v.at[b2], gsem[b2])
        sd[U - 2].wait()
        sd[U - 1].wait()
        return carry

    lax.fori_loop(0, NCH // U, step, 0)

    plsc.subcore_barrier()
    base = jnp.where(s == NSUB - 1, N - OUT_PT, s * OUT_PT)
    pltpu.sync_copy(acc_sh.at[pl.ds(base, OUT_PT)],
                    out_hbm.at[pl.ds(base, OUT_PT), c])


_agg = pl.kernel(
    _agg_body,
    out_type=jax.ShapeDtypeStruct((N, NCORE, H), jnp.float32),
    mesh=plsc.VectorSubcoreMesh(core_axis_name="c", subcore_axis_name="s"),
    scratch_types=[
        pltpu.VMEM((2, U, CH), jnp.int32),
        pltpu.VMEM((NB_ROWS, CH, H), jnp.float32),
        pltpu.VMEM_SHARED((N_PAD, H), jnp.float32),
        pltpu.SemaphoreType.DMA,
        pltpu.SemaphoreType.DMA,
        pltpu.SemaphoreType.DMA,
        pltpu.SemaphoreType.DMA,
        pltpu.SemaphoreType.DMA,
        pltpu.SemaphoreType.DMA,
    ],
)


def kernel(feature, edge_index, W1, b1, W2, b2):
    src = edge_index[0].astype(jnp.int32)
    dst = edge_index[1].astype(jnp.int32)

    pad = E_PAD - src.shape[0]
    pad_idx = jnp.arange(pad, dtype=jnp.int32)
    src_p = jnp.concatenate([src, (pad_idx * 41) % N])
    dst_p = jnp.concatenate([dst, N + (pad_idx % (N_PAD - N))]).astype(jnp.int32)
    src_r = (2 * src_p).reshape(NSUB, NCH // U, U, CH)
    dst_r = dst_p.reshape(NSUB, NCH // U, U, CH)

    b1_blk = jnp.broadcast_to(b1.reshape(NCORE, 1, H), (NCORE, CH, H))
    b2_blk = jnp.broadcast_to(b2.reshape(NCORE, 1, H), (NCORE, CH, H))

    t1 = _mm1(feature, W1)
    h1 = _agg(t1.reshape(NCORE * N, H), src_r, dst_r, b1_blk)
    t2 = _mm2(h1, W2)
    h2 = _agg(t2.reshape(NCORE * N, H), src_r, dst_r, b2_blk)
    return h2.reshape(N, D)


# speedup vs baseline: 1.0079x; 1.0079x over previous
import jax
import jax.numpy as jnp
from jax import lax
from jax.experimental import pallas as pl
from jax.experimental.pallas import tpu as pltpu
from jax.experimental.pallas import tpu_sc as plsc

N = 10000
D = 256
H = 128
NCORE = 2
NSUB = 16
CH = 112
NCH = 90
U = 6
NB_ROWS = 3
E_PAD = NSUB * NCH * CH
N_PAD = 10112
ROWS_PT = N_PAD // NSUB
OUT_PT = 632


def _mm1_body(x_ref, w_ref, o_ref):
    x = x_ref[...]
    o_ref[:, 0, :] = jnp.dot(x, w_ref[:, :H], preferred_element_type=jnp.float32)
    o_ref[:, 1, :] = jnp.dot(x, w_ref[:, H:], preferred_element_type=jnp.float32)


def _mm2_body(h_ref, w_ref, o_ref):
    a0 = jnp.maximum(h_ref[:, 0, :], 0.0)
    a1 = jnp.maximum(h_ref[:, 1, :], 0.0)
    o_ref[:, 0, :] = (
        jnp.dot(a0, w_ref[:H, :H], preferred_element_type=jnp.float32)
        + jnp.dot(a1, w_ref[H:, :H], preferred_element_type=jnp.float32))
    o_ref[:, 1, :] = (
        jnp.dot(a0, w_ref[:H, H:], preferred_element_type=jnp.float32)
        + jnp.dot(a1, w_ref[H:, H:], preferred_element_type=jnp.float32))


_RB = 1000
_NB = N // _RB

_mm1 = pl.pallas_call(
    _mm1_body,
    grid=(_NB,),
    in_specs=[
        pl.BlockSpec((_RB, D), lambda i: (i, 0)),
        pl.BlockSpec((D, D), lambda i: (0, 0)),
    ],
    out_specs=pl.BlockSpec((_RB, NCORE, H), lambda i: (i, 0, 0)),
    out_shape=jax.ShapeDtypeStruct((N, NCORE, H), jnp.float32),
)

_mm2 = pl.pallas_call(
    _mm2_body,
    grid=(_NB,),
    in_specs=[
        pl.BlockSpec((_RB, NCORE, H), lambda i: (i, 0, 0)),
        pl.BlockSpec((D, D), lambda i: (0, 0)),
    ],
    out_specs=pl.BlockSpec((_RB, NCORE, H), lambda i: (i, 0, 0)),
    out_shape=jax.ShapeDtypeStruct((N, NCORE, H), jnp.float32),
)


def _agg_body(t_hbm, src_hbm, dst_hbm, b_hbm, out_hbm,
              idx_v, rows_v, acc_sh, gsem0, gsem1, gsem2, ssem0, ssem1, ssem2):
    gsem = (gsem0, gsem1, gsem2)
    ssem = (ssem0, ssem1, ssem2)
    c = lax.axis_index("c")
    s = lax.axis_index("s")

    pltpu.sync_copy(b_hbm.at[c], rows_v.at[0])
    for k in range(ROWS_PT // CH):
        pltpu.sync_copy(rows_v.at[0],
                        acc_sh.at[pl.ds(s * ROWS_PT + k * CH, CH)])
    _rem = ROWS_PT % CH
    if _rem:
        pltpu.sync_copy(
            rows_v.at[0, pl.ds(0, _rem)],
            acc_sh.at[pl.ds(s * ROWS_PT + (ROWS_PT // CH) * CH, _rem)])
    plsc.subcore_barrier()

    def step(jj, carry):
        pltpu.sync_copy(src_hbm.at[s, jj], idx_v.at[0])
        pltpu.sync_copy(dst_hbm.at[s, jj], idx_v.at[1])

        @pl.when(c == 1)
        def _bump():
            for u in range(U):
                for k in range(CH // 16):
                    idx_v[0, u, pl.ds(k * 16, 16)] = (
                        idx_v[0, u, pl.ds(k * 16, 16)] + 1)

        gd = [None] * U
        sd = [None] * U
        gd[0] = pltpu.async_copy(t_hbm.at[idx_v.at[0, 0]], rows_v.at[0],
                                 gsem[0])
        gd[1] = pltpu.async_copy(t_hbm.at[idx_v.at[0, 1]], rows_v.at[1],
                                 gsem[1])
        for u in range(U):
            b = u % NB_ROWS
            gd[u].wait()
            sd[u] = pltpu.async_copy(rows_v.at[b], acc_sh.at[idx_v.at[1, u]],
                                     ssem[b], add=True)
            if u + 2 < U:
                if u >= 1:
                    sd[u - 1].wait()
                b2 = (u + 2) % NB_ROWS
                gd[u + 2] = pltpu.async_copy(
                    t_hbm.at[idx_v.at[0, u + 2]], rows_v.at[b2], gsem[b2])
        sd[U - 3].wait()
        sd[U - 2].wait()
        sd[U - 1].wait()
        return carry

    lax.fori_loop(0, NCH // U, step, 0)

    plsc.subcore_barrier()
    base = jnp.where(s == NSUB - 1, N - OUT_PT, s * OUT_PT)
    pltpu.sync_copy(acc_sh.at[pl.ds(base, OUT_PT)],
                    out_hbm.at[pl.ds(base, OUT_PT), c])


_agg = pl.kernel(
    _agg_body,
    out_type=jax.ShapeDtypeStruct((N, NCORE, H), jnp.float32),
    mesh=plsc.VectorSubcoreMesh(core_axis_name="c", subcore_axis_name="s"),
    scratch_types=[
        pltpu.VMEM((2, U, CH), jnp.int32),
        pltpu.VMEM((NB_ROWS, CH, H), jnp.float32),
        pltpu.VMEM_SHARED((N_PAD, H), jnp.float32),
        pltpu.SemaphoreType.DMA,
        pltpu.SemaphoreType.DMA,
        pltpu.SemaphoreType.DMA,
        pltpu.SemaphoreType.DMA,
        pltpu.SemaphoreType.DMA,
        pltpu.SemaphoreType.DMA,
    ],
)


def kernel(feature, edge_index, W1, b1, W2, b2):
    src = edge_index[0].astype(jnp.int32)
    dst = edge_index[1].astype(jnp.int32)

    pad = E_PAD - src.shape[0]
    pad_idx = jnp.arange(pad, dtype=jnp.int32)
    src_p = jnp.concatenate([src, (pad_idx * 41) % N])
    dst_p = jnp.concatenate([dst, N + (pad_idx % (N_PAD - N))]).astype(jnp.int32)
    src_r = (2 * src_p).reshape(NSUB, NCH // U, U, CH)
    dst_r = dst_p.reshape(NSUB, NCH // U, U, CH)

    b1_blk = jnp.broadcast_to(b1.reshape(NCORE, 1, H), (NCORE, CH, H))
    b2_blk = jnp.broadcast_to(b2.reshape(NCORE, 1, H), (NCORE, CH, H))

    t1 = _mm1(feature, W1)
    h1 = _agg(t1.reshape(NCORE * N, H), src_r, dst_r, b1_blk)
    t2 = _mm2(h1, W2)
    h2 = _agg(t2.reshape(NCORE * N, H), src_r, dst_r, b2_blk)
    return h2.reshape(N, D)


# speedup vs baseline: 1.2284x; 1.2188x over previous
import jax
import jax.numpy as jnp
from jax import lax
from jax.experimental import pallas as pl
from jax.experimental.pallas import tpu as pltpu
from jax.experimental.pallas import tpu_sc as plsc

N = 10000
D = 256
H = 128
NCORE = 2
NSUB = 16
CH = 112
NCH = 90
U = 18
NB_ROWS = 3
E_PAD = NSUB * NCH * CH
N_PAD = 10112
ROWS_PT = N_PAD // NSUB
OUT_PT = 632


def _mm1_body(x_ref, w_ref, o_ref):
    x = x_ref[...]
    o_ref[:, 0, :] = jnp.dot(x, w_ref[:, :H], preferred_element_type=jnp.float32)
    o_ref[:, 1, :] = jnp.dot(x, w_ref[:, H:], preferred_element_type=jnp.float32)


def _mm2_body(h_ref, w_ref, o_ref):
    a0 = jnp.maximum(h_ref[:, 0, :], 0.0)
    a1 = jnp.maximum(h_ref[:, 1, :], 0.0)
    o_ref[:, 0, :] = (
        jnp.dot(a0, w_ref[:H, :H], preferred_element_type=jnp.float32)
        + jnp.dot(a1, w_ref[H:, :H], preferred_element_type=jnp.float32))
    o_ref[:, 1, :] = (
        jnp.dot(a0, w_ref[:H, H:], preferred_element_type=jnp.float32)
        + jnp.dot(a1, w_ref[H:, H:], preferred_element_type=jnp.float32))


_RB = 1000
_NB = N // _RB

_mm1 = pl.pallas_call(
    _mm1_body,
    grid=(_NB,),
    in_specs=[
        pl.BlockSpec((_RB, D), lambda i: (i, 0)),
        pl.BlockSpec((D, D), lambda i: (0, 0)),
    ],
    out_specs=pl.BlockSpec((_RB, NCORE, H), lambda i: (i, 0, 0)),
    out_shape=jax.ShapeDtypeStruct((N, NCORE, H), jnp.float32),
)

_mm2 = pl.pallas_call(
    _mm2_body,
    grid=(_NB,),
    in_specs=[
        pl.BlockSpec((_RB, NCORE, H), lambda i: (i, 0, 0)),
        pl.BlockSpec((D, D), lambda i: (0, 0)),
    ],
    out_specs=pl.BlockSpec((_RB, NCORE, H), lambda i: (i, 0, 0)),
    out_shape=jax.ShapeDtypeStruct((N, NCORE, H), jnp.float32),
)


def _agg_body(t_hbm, src_hbm, dst_hbm, b_hbm, out_hbm,
              idx_v, rows_v, acc_sh, gsem0, gsem1, gsem2):
    gsem = (gsem0, gsem1, gsem2)
    c = lax.axis_index("c")
    s = lax.axis_index("s")

    pltpu.sync_copy(b_hbm.at[c], rows_v.at[0])
    for k in range(ROWS_PT // CH):
        pltpu.sync_copy(rows_v.at[0],
                        acc_sh.at[pl.ds(s * ROWS_PT + k * CH, CH)])
    _rem = ROWS_PT % CH
    if _rem:
        pltpu.sync_copy(
            rows_v.at[0, pl.ds(0, _rem)],
            acc_sh.at[pl.ds(s * ROWS_PT + (ROWS_PT // CH) * CH, _rem)])
    plsc.subcore_barrier()

    def step(jj, carry):
        pltpu.sync_copy(src_hbm.at[s, jj], idx_v.at[0])
        pltpu.sync_copy(dst_hbm.at[s, jj], idx_v.at[1])

        @pl.when(c == 1)
        def _bump():
            for u in range(U):
                for k in range(CH // 16):
                    idx_v[0, u, pl.ds(k * 16, 16)] = (
                        idx_v[0, u, pl.ds(k * 16, 16)] + 1)

        gd = [None] * U
        gd[0] = pltpu.async_copy(t_hbm.at[idx_v.at[0, 0]], rows_v.at[0],
                                 gsem[0])
        gd[1] = pltpu.async_copy(t_hbm.at[idx_v.at[0, 1]], rows_v.at[1],
                                 gsem[1])
        for u in range(U):
            b = u % NB_ROWS
            gd[u].wait()
            if u + 2 < U:
                b2 = (u + 2) % NB_ROWS
                gd[u + 2] = pltpu.async_copy(
                    t_hbm.at[idx_v.at[0, u + 2]], rows_v.at[b2], gsem[b2])
            pltpu.sync_copy(rows_v.at[b], acc_sh.at[idx_v.at[1, u]], add=True)
        return carry

    lax.fori_loop(0, NCH // U, step, 0)

    plsc.subcore_barrier()
    base = jnp.where(s == NSUB - 1, N - OUT_PT, s * OUT_PT)
    pltpu.sync_copy(acc_sh.at[pl.ds(base, OUT_PT)],
                    out_hbm.at[pl.ds(base, OUT_PT), c])


_agg = pl.kernel(
    _agg_body,
    out_type=jax.ShapeDtypeStruct((N, NCORE, H), jnp.float32),
    mesh=plsc.VectorSubcoreMesh(core_axis_name="c", subcore_axis_name="s"),
    scratch_types=[
        pltpu.VMEM((2, U, CH), jnp.int32),
        pltpu.VMEM((NB_ROWS, CH, H), jnp.float32),
        pltpu.VMEM_SHARED((N_PAD, H), jnp.float32),
        pltpu.SemaphoreType.DMA,
        pltpu.SemaphoreType.DMA,
        pltpu.SemaphoreType.DMA,
    ],
)


def kernel(feature, edge_index, W1, b1, W2, b2):
    src = edge_index[0].astype(jnp.int32)
    dst = edge_index[1].astype(jnp.int32)

    pad = E_PAD - src.shape[0]
    pad_idx = jnp.arange(pad, dtype=jnp.int32)
    src_p = jnp.concatenate([src, (pad_idx * 41) % N])
    dst_p = jnp.concatenate([dst, N + (pad_idx % (N_PAD - N))]).astype(jnp.int32)
    src_r = (2 * src_p).reshape(NSUB, NCH // U, U, CH)
    dst_r = dst_p.reshape(NSUB, NCH // U, U, CH)

    b1_blk = jnp.broadcast_to(b1.reshape(NCORE, 1, H), (NCORE, CH, H))
    b2_blk = jnp.broadcast_to(b2.reshape(NCORE, 1, H), (NCORE, CH, H))

    t1 = _mm1(feature, W1)
    h1 = _agg(t1.reshape(NCORE * N, H), src_r, dst_r, b1_blk)
    t2 = _mm2(h1, W2)
    h2 = _agg(t2.reshape(NCORE * N, H), src_r, dst_r, b2_blk)
    return h2.reshape(N, D)
